# baseline (speedup 1.0000x reference)
import jax
import jax.numpy as jnp
from jax.experimental import pallas as pl
from jax.experimental.pallas import tpu as pltpu

_B = 16
_N = 4096
_C = 4
_CB = _B // _C
_AR = _B * _N // 128


def _body(x_hbm, out_hbm, cnt_hbm, assoc_hbm,
          in_v, out_v, cnt_v, assoc_v, in_sem, out_sem, aux_sem):
    in_cp = [
        pltpu.make_async_copy(
            x_hbm.at[pl.ds(c * _CB, _CB)], in_v.at[pl.ds(c * _CB, _CB)],
            in_sem.at[c])
        for c in range(_C)
    ]
    for cp in in_cp:
        cp.start()

    r = jax.lax.broadcasted_iota(jnp.int32, (_AR, 128), 0)
    assoc_v[...] = r >> 5
    cnt_v[...] = jnp.full((16,), _N, dtype=jnp.int32)
    assoc_cp = pltpu.make_async_copy(assoc_v, assoc_hbm, aux_sem.at[0])
    cnt_cp = pltpu.make_async_copy(cnt_v, cnt_hbm, aux_sem.at[1])
    assoc_cp.start()
    cnt_cp.start()

    coord = jax.lax.broadcasted_iota(jnp.int32, (_CB, 4, _N), 1)
    out_cp = []
    for c in range(_C):
        in_cp[c].wait()
        sl = pl.ds(c * _CB, _CB)
        y = in_v[sl] * 0.5
        out_v[sl] = jnp.where(coord < 2, jnp.floor(y), jnp.ceil(y))
        cp = pltpu.make_async_copy(out_v.at[sl], out_hbm.at[sl], out_sem.at[c])
        cp.start()
        out_cp.append(cp)

    for cp in out_cp:
        cp.wait()
    assoc_cp.wait()
    cnt_cp.wait()


_tc_call = pl.pallas_call(
    _body,
    in_specs=[pl.BlockSpec(memory_space=pl.ANY)],
    out_specs=[
        pl.BlockSpec(memory_space=pl.ANY),
        pl.BlockSpec(memory_space=pl.ANY),
        pl.BlockSpec(memory_space=pl.ANY),
    ],
    out_shape=[
        jax.ShapeDtypeStruct((_B, 4, _N), jnp.float32),
        jax.ShapeDtypeStruct((16,), jnp.int32),
        jax.ShapeDtypeStruct((_AR, 128), jnp.int32),
    ],
    scratch_shapes=[
        pltpu.VMEM((_B, 4, _N), jnp.float32),
        pltpu.VMEM((_B, 4, _N), jnp.float32),
        pltpu.VMEM((16,), jnp.int32),
        pltpu.VMEM((_AR, 128), jnp.int32),
        pltpu.SemaphoreType.DMA((_C,)),
        pltpu.SemaphoreType.DMA((_C,)),
        pltpu.SemaphoreType.DMA((2,)),
    ],
)


def kernel(bbox_batch):
    xt = bbox_batch.transpose(0, 2, 1)
    out_t, cnt, assoc = _tc_call(xt)
    return (
        out_t.transpose(0, 2, 1).reshape(_B * _N, 4),
        cnt,
        assoc.reshape(_B * _N),
    )
